# baseline (speedup 1.0000x reference)
import jax
import jax.numpy as jnp
from jax import lax
from jax.experimental import pallas as pl
from jax.experimental.pallas import tpu as pltpu
from jax.experimental.pallas import tpu_sc as plsc

_N = 10000
_E = 320000
_R = 8
_D_IN = 128
_D_HID = 64
_D_OUT = 32

_NC = 2
_NS = 16
_NW = _NC * _NS
_CHUNK = 128
_NCH = 80
_EPT = _NCH * _CHUNK
_E_PAD = _EPT * _NW
_N_ROWS = 10240
_ZR = _N_ROWS // _NS
_OR = _N_ROWS // _NS
_DEG_W = 8

_BN = 1000


def _make_sc_agg(d, k_rows, with_deg):
  mesh = plsc.VectorSubcoreMesh(core_axis_name="c", subcore_axis_name="s")
  n_steps = _NCH // k_rows
  assert _NCH % k_rows == 0
  idx_rows = (n_steps + 1) * k_rows
  ce = k_rows * _CHUNK

  if with_deg:
    out_type = [jax.ShapeDtypeStruct((_NC, _N_ROWS, d), jnp.float32),
                jax.ShapeDtypeStruct((_NC, _N_ROWS, _DEG_W), jnp.float32)]
  else:
    out_type = jax.ShapeDtypeStruct((_NC, _N_ROWS, d), jnp.float32)
  scratch = [
      pltpu.VMEM(((n_steps + 1) * k_rows, _CHUNK), jnp.int32),
      pltpu.VMEM((_NCH, _CHUNK), jnp.int32),
      pltpu.VMEM((ce, d), jnp.float32),
      pltpu.VMEM((ce, d), jnp.float32),
      pltpu.VMEM_SHARED((_N_ROWS, d), jnp.float32),
      pltpu.SemaphoreType.DMA,
      pltpu.SemaphoreType.DMA,
  ]
  if with_deg:
    scratch += [
        pltpu.VMEM((_CHUNK, _DEG_W), jnp.float32),
        pltpu.VMEM_SHARED((_N_ROWS, _DEG_W), jnp.float32),
    ]

  def body(*refs):
    if with_deg:
      (gidx_hbm, dst_hbm, table_hbm, zrow_hbm, zdeg_hbm, ones_hbm,
       out_hbm, deg_hbm,
       gidx_v, dst_v, buf0, buf1, agg_sh, sem0, sem1, ones_v, deg_sh) = refs
    else:
      (gidx_hbm, dst_hbm, table_hbm, zrow_hbm,
       out_hbm,
       gidx_v, dst_v, buf0, buf1, agg_sh, sem0, sem1) = refs
    c = lax.axis_index("c")
    s = lax.axis_index("s")
    wid = c * _NS + s
    bufs = (buf0, buf1)
    sems = (sem0, sem1)

    pltpu.sync_copy(zrow_hbm, agg_sh.at[pl.ds(s * _ZR, _ZR)])
    if with_deg:
      pltpu.sync_copy(zdeg_hbm, deg_sh.at[pl.ds(s * _ZR, _ZR)])
      pltpu.sync_copy(ones_hbm, ones_v)
    pltpu.sync_copy(gidx_hbm.at[wid], gidx_v)
    pltpu.sync_copy(dst_hbm.at[wid], dst_v)
    plsc.subcore_barrier()

    def fire(chunk, buf, sem):
      for j in range(k_rows):
        pltpu.async_copy(table_hbm.at[gidx_v.at[chunk * k_rows + j]],
                         buf.at[pl.ds(j * _CHUNK, _CHUNK)], sem)

    def drain(buf, sem):
      for j in range(k_rows):
        pltpu.make_async_copy(table_hbm.at[gidx_v.at[0]],
                              buf.at[pl.ds(j * _CHUNK, _CHUNK)], sem).wait()

    def scatter(chunk, buf):
      for j in range(k_rows):
        idx = dst_v.at[chunk * k_rows + j]
        pltpu.sync_copy(buf.at[pl.ds(j * _CHUNK, _CHUNK)],
                        agg_sh.at[idx], add=True)
        if with_deg:
          pltpu.sync_copy(ones_v, deg_sh.at[idx], add=True)

    fire(0, bufs[0], sems[0])

    def step(i, carry):
      fire(2 * i + 1, bufs[1], sems[1])
      drain(bufs[0], sems[0])
      scatter(2 * i, bufs[0])
      fire(2 * i + 2, bufs[0], sems[0])
      drain(bufs[1], sems[1])
      scatter(2 * i + 1, bufs[1])
      return carry
    lax.fori_loop(0, n_steps // 2, step, 0)
    drain(bufs[0], sems[0])

    plsc.subcore_barrier()
    pltpu.sync_copy(agg_sh.at[pl.ds(s * _OR, _OR)],
                    out_hbm.at[c].at[pl.ds(s * _OR, _OR)])
    if with_deg:
      pltpu.sync_copy(deg_sh.at[pl.ds(s * _OR, _OR)],
                      deg_hbm.at[c].at[pl.ds(s * _OR, _OR)])

  return pl.kernel(
      body, out_type=out_type, mesh=mesh, scratch_types=scratch,
      compiler_params=pltpu.CompilerParams(use_tc_tiling_on_sc=False))


def _tc_a_body(x_ref, wc_ref, ws_ref, xw_ref, sf_ref):
  xb = x_ref[...]
  xw_ref[...] = jnp.dot(xb, wc_ref[...], preferred_element_type=jnp.float32)
  sf_ref[...] = jnp.dot(xb, ws_ref[...], preferred_element_type=jnp.float32)


_tc_a = pl.pallas_call(
    _tc_a_body,
    grid=(_N // _BN,),
    in_specs=[
        pl.BlockSpec((_BN, _D_IN), lambda i: (i, 0)),
        pl.BlockSpec((_D_IN, _R * _D_HID), lambda i: (0, 0)),
        pl.BlockSpec((_D_IN, _D_HID), lambda i: (0, 0)),
    ],
    out_specs=[
        pl.BlockSpec((_BN, _R * _D_HID), lambda i: (i, 0)),
        pl.BlockSpec((_BN, _D_HID), lambda i: (i, 0)),
    ],
    out_shape=[
        jax.ShapeDtypeStruct((_N, _R * _D_HID), jnp.float32),
        jax.ShapeDtypeStruct((_N, _D_HID), jnp.float32),
    ],
)


def _tc_b_body(p0_ref, p1_ref, d0_ref, d1_ref, s1_ref, wc_ref, ws_ref,
               xw_ref, sf_ref):
  deg = jnp.maximum(d0_ref[:, 0:1] + d1_ref[:, 0:1], 1.0)
  h = jnp.maximum((p0_ref[...] + p1_ref[...]) / deg + s1_ref[...], 0.0)
  xw_ref[...] = jnp.dot(h, wc_ref[...], preferred_element_type=jnp.float32)
  sf_ref[...] = jnp.dot(h, ws_ref[...], preferred_element_type=jnp.float32)


_tc_b = pl.pallas_call(
    _tc_b_body,
    grid=(_N // _BN,),
    in_specs=[
        pl.BlockSpec((_BN, _D_HID), lambda i: (i, 0)),
        pl.BlockSpec((_BN, _D_HID), lambda i: (i, 0)),
        pl.BlockSpec((_BN, _DEG_W), lambda i: (i, 0)),
        pl.BlockSpec((_BN, _DEG_W), lambda i: (i, 0)),
        pl.BlockSpec((_BN, _D_HID), lambda i: (i, 0)),
        pl.BlockSpec((_D_HID, _R * _D_OUT), lambda i: (0, 0)),
        pl.BlockSpec((_D_HID, _D_OUT), lambda i: (0, 0)),
    ],
    out_specs=[
        pl.BlockSpec((_BN, _R * _D_OUT), lambda i: (i, 0)),
        pl.BlockSpec((_BN, _D_OUT), lambda i: (i, 0)),
    ],
    out_shape=[
        jax.ShapeDtypeStruct((_N, _R * _D_OUT), jnp.float32),
        jax.ShapeDtypeStruct((_N, _D_OUT), jnp.float32),
    ],
)


def _tc_c_body(q0_ref, q1_ref, d0_ref, d1_ref, s2_ref, out_ref):
  deg = jnp.maximum(d0_ref[:, 0:1] + d1_ref[:, 0:1], 1.0)
  out_ref[...] = (q0_ref[...] + q1_ref[...]) / deg + s2_ref[...]


_tc_c = pl.pallas_call(
    _tc_c_body,
    grid=(_N // _BN,),
    in_specs=[
        pl.BlockSpec((_BN, _D_OUT), lambda i: (i, 0)),
        pl.BlockSpec((_BN, _D_OUT), lambda i: (i, 0)),
        pl.BlockSpec((_BN, _DEG_W), lambda i: (i, 0)),
        pl.BlockSpec((_BN, _DEG_W), lambda i: (i, 0)),
        pl.BlockSpec((_BN, _D_OUT), lambda i: (i, 0)),
    ],
    out_specs=pl.BlockSpec((_BN, _D_OUT), lambda i: (i, 0)),
    out_shape=jax.ShapeDtypeStruct((_N, _D_OUT), jnp.float32),
)


def kernel(x, edge_index, edge_type, W1_rel, W1_self, W2_rel, W2_self):
  src, dst = edge_index[0], edge_index[1]
  pad = _E_PAD - _E
  gidx = jnp.concatenate(
      [src * _R + edge_type, jnp.zeros((pad,), jnp.int32)]
  ).reshape(_NW, _NCH, _CHUNK)
  dstp = jnp.concatenate(
      [dst, jnp.full((pad,), _N, jnp.int32)]
  ).reshape(_NW, _NCH, _CHUNK)
  k1, k2 = 2, 8
  gidx1 = jnp.concatenate(
      [gidx, jnp.zeros((_NW, k1, _CHUNK), jnp.int32)], axis=1
  )
  gidx2 = jnp.concatenate(
      [gidx, jnp.zeros((_NW, k2, _CHUNK), jnp.int32)], axis=1
  )

  wc1 = W1_rel.transpose(1, 0, 2).reshape(_D_IN, _R * _D_HID)
  wc2 = W2_rel.transpose(1, 0, 2).reshape(_D_HID, _R * _D_OUT)

  zrow1 = jnp.zeros((_ZR, _D_HID), jnp.float32)
  zrow2 = jnp.zeros((_ZR, _D_OUT), jnp.float32)
  zdeg = jnp.zeros((_ZR, _DEG_W), jnp.float32)
  ones = jnp.ones((_CHUNK, _DEG_W), jnp.float32)

  xw1, self1 = _tc_a(x, wc1, W1_self)
  agg1, deg = _make_sc_agg(_D_HID, k1, True)(
      gidx1, dstp, xw1.reshape(_N * _R, _D_HID), zrow1, zdeg, ones)
  xw2, self2 = _tc_b(agg1[0], agg1[1], deg[0], deg[1], self1, wc2, W2_self)
  agg2 = _make_sc_agg(_D_OUT, k2, False)(
      gidx2, dstp, xw2.reshape(_N * _R, _D_OUT), zrow2)
  out = _tc_c(agg2[0], agg2[1], deg[0], deg[1], self2)
  return out


# speedup vs baseline: 2.2062x; 2.2062x over previous
import jax
import jax.numpy as jnp
from jax import lax
from jax.experimental import pallas as pl
from jax.experimental.pallas import tpu as pltpu
from jax.experimental.pallas import tpu_sc as plsc

_N = 10000
_E = 320000
_R = 8
_D_IN = 128
_D_HID = 64
_D_OUT = 32

_NC = 2
_NS = 16
_NW = _NC * _NS
_CHUNK = 128
_NCH = 80
_EPT = _NCH * _CHUNK
_E_PAD = _EPT * _NW
_N_ROWS = 10240
_ZR = _N_ROWS // _NS
_OR = _N_ROWS // _NS
_DEG_W = 8

_BN = 2000


def _make_sc_agg(d, with_deg):
  mesh = plsc.VectorSubcoreMesh(core_axis_name="c", subcore_axis_name="s")
  if with_deg:
    out_type = [jax.ShapeDtypeStruct((_NC, _N_ROWS, d), jnp.bfloat16),
                jax.ShapeDtypeStruct((_NC, _N_ROWS, _DEG_W), jnp.float32)]
  else:
    out_type = jax.ShapeDtypeStruct((_NC, _N_ROWS, d), jnp.bfloat16)
  scratch = [
      pltpu.VMEM((_NCH, _CHUNK), jnp.int32),
      pltpu.VMEM((_NCH, _CHUNK), jnp.int32),
      pltpu.VMEM((_CHUNK, d), jnp.bfloat16),
      pltpu.VMEM_SHARED((_N_ROWS, d), jnp.bfloat16),
      pltpu.SemaphoreType.DMA,
  ]
  if with_deg:
    scratch += [
        pltpu.VMEM((_CHUNK, _DEG_W), jnp.float32),
        pltpu.VMEM_SHARED((_N_ROWS, _DEG_W), jnp.float32),
    ]

  def body(*refs):
    if with_deg:
      (gidx_hbm, dst_hbm, table_hbm, zrow_hbm, zdeg_hbm, ones_hbm,
       out_hbm, deg_hbm,
       gidx_v, dst_v, rows_v, agg_sh, sem, ones_v, deg_sh) = refs
    else:
      (gidx_hbm, dst_hbm, table_hbm, zrow_hbm,
       out_hbm,
       gidx_v, dst_v, rows_v, agg_sh, sem) = refs
    c = lax.axis_index("c")
    s = lax.axis_index("s")
    wid = c * _NS + s

    pltpu.sync_copy(zrow_hbm, agg_sh.at[pl.ds(s * _ZR, _ZR)])
    if with_deg:
      pltpu.sync_copy(zdeg_hbm, deg_sh.at[pl.ds(s * _ZR, _ZR)])
      pltpu.sync_copy(ones_hbm, ones_v)
    pltpu.sync_copy(gidx_hbm.at[wid], gidx_v)
    pltpu.sync_copy(dst_hbm.at[wid], dst_v)
    plsc.subcore_barrier()

    def step(i, carry):
      pltpu.async_copy(table_hbm.at[gidx_v.at[i]], rows_v, sem).wait()
      pltpu.sync_copy(rows_v, agg_sh.at[dst_v.at[i]], add=True)
      if with_deg:
        pltpu.sync_copy(ones_v, deg_sh.at[dst_v.at[i]], add=True)
      return carry
    lax.fori_loop(0, _NCH, step, 0)

    plsc.subcore_barrier()
    pltpu.sync_copy(agg_sh.at[pl.ds(s * _OR, _OR)],
                    out_hbm.at[c].at[pl.ds(s * _OR, _OR)])
    if with_deg:
      pltpu.sync_copy(deg_sh.at[pl.ds(s * _OR, _OR)],
                      deg_hbm.at[c].at[pl.ds(s * _OR, _OR)])

  return pl.kernel(
      body, out_type=out_type, mesh=mesh, scratch_types=scratch,
      compiler_params=pltpu.CompilerParams(use_tc_tiling_on_sc=False))


def _tc_a_body(x_ref, wc_ref, ws_ref, xw_ref, sf_ref):
  xb = x_ref[...]
  xw = jnp.dot(xb, wc_ref[...], preferred_element_type=jnp.float32)
  xw_ref[...] = xw.astype(jnp.bfloat16)
  sf_ref[...] = jnp.dot(xb, ws_ref[...], preferred_element_type=jnp.float32)


_tc_a = pl.pallas_call(
    _tc_a_body,
    grid=(_N // _BN,),
    in_specs=[
        pl.BlockSpec((_BN, _D_IN), lambda i: (i, 0)),
        pl.BlockSpec((_D_IN, _R * _D_HID), lambda i: (0, 0)),
        pl.BlockSpec((_D_IN, _D_HID), lambda i: (0, 0)),
    ],
    out_specs=[
        pl.BlockSpec((_BN, _R * _D_HID), lambda i: (i, 0)),
        pl.BlockSpec((_BN, _D_HID), lambda i: (i, 0)),
    ],
    out_shape=[
        jax.ShapeDtypeStruct((_N, _R * _D_HID), jnp.bfloat16),
        jax.ShapeDtypeStruct((_N, _D_HID), jnp.float32),
    ],
)


def _tc_b_body(p0_ref, p1_ref, d0_ref, d1_ref, s1_ref, wc_ref, ws_ref,
               xw_ref, sf_ref):
  deg = jnp.maximum(d0_ref[:, 0:1] + d1_ref[:, 0:1], 1.0)
  p = p0_ref[...].astype(jnp.float32) + p1_ref[...].astype(jnp.float32)
  h = jnp.maximum(p / deg + s1_ref[...], 0.0)
  xw = jnp.dot(h, wc_ref[...], preferred_element_type=jnp.float32)
  xw_ref[...] = xw.astype(jnp.bfloat16)
  sf_ref[...] = jnp.dot(h, ws_ref[...], preferred_element_type=jnp.float32)


_tc_b = pl.pallas_call(
    _tc_b_body,
    grid=(_N // _BN,),
    in_specs=[
        pl.BlockSpec((_BN, _D_HID), lambda i: (i, 0)),
        pl.BlockSpec((_BN, _D_HID), lambda i: (i, 0)),
        pl.BlockSpec((_BN, _DEG_W), lambda i: (i, 0)),
        pl.BlockSpec((_BN, _DEG_W), lambda i: (i, 0)),
        pl.BlockSpec((_BN, _D_HID), lambda i: (i, 0)),
        pl.BlockSpec((_D_HID, _R * _D_OUT), lambda i: (0, 0)),
        pl.BlockSpec((_D_HID, _D_OUT), lambda i: (0, 0)),
    ],
    out_specs=[
        pl.BlockSpec((_BN, _R * _D_OUT), lambda i: (i, 0)),
        pl.BlockSpec((_BN, _D_OUT), lambda i: (i, 0)),
    ],
    out_shape=[
        jax.ShapeDtypeStruct((_N, _R * _D_OUT), jnp.bfloat16),
        jax.ShapeDtypeStruct((_N, _D_OUT), jnp.float32),
    ],
)


def _tc_c_body(q0_ref, q1_ref, d0_ref, d1_ref, s2_ref, out_ref):
  deg = jnp.maximum(d0_ref[:, 0:1] + d1_ref[:, 0:1], 1.0)
  q = q0_ref[...].astype(jnp.float32) + q1_ref[...].astype(jnp.float32)
  out_ref[...] = q / deg + s2_ref[...]


_tc_c = pl.pallas_call(
    _tc_c_body,
    grid=(_N // _BN,),
    in_specs=[
        pl.BlockSpec((_BN, _D_OUT), lambda i: (i, 0)),
        pl.BlockSpec((_BN, _D_OUT), lambda i: (i, 0)),
        pl.BlockSpec((_BN, _DEG_W), lambda i: (i, 0)),
        pl.BlockSpec((_BN, _DEG_W), lambda i: (i, 0)),
        pl.BlockSpec((_BN, _D_OUT), lambda i: (i, 0)),
    ],
    out_specs=pl.BlockSpec((_BN, _D_OUT), lambda i: (i, 0)),
    out_shape=jax.ShapeDtypeStruct((_N, _D_OUT), jnp.float32),
)


def kernel(x, edge_index, edge_type, W1_rel, W1_self, W2_rel, W2_self):
  src, dst = edge_index[0], edge_index[1]
  pad = _E_PAD - _E
  gidx = jnp.concatenate(
      [src * _R + edge_type, jnp.zeros((pad,), jnp.int32)]
  ).reshape(_NW, _NCH, _CHUNK)
  dstp = jnp.concatenate(
      [dst, jnp.full((pad,), _N, jnp.int32)]
  ).reshape(_NW, _NCH, _CHUNK)

  wc1 = W1_rel.transpose(1, 0, 2).reshape(_D_IN, _R * _D_HID)
  wc2 = W2_rel.transpose(1, 0, 2).reshape(_D_HID, _R * _D_OUT)

  zrow1 = jnp.zeros((_ZR, _D_HID), jnp.bfloat16)
  zrow2 = jnp.zeros((_ZR, _D_OUT), jnp.bfloat16)
  zdeg = jnp.zeros((_ZR, _DEG_W), jnp.float32)
  ones = jnp.ones((_CHUNK, _DEG_W), jnp.float32)

  xw1, self1 = _tc_a(x, wc1, W1_self)
  agg1, deg = _make_sc_agg(_D_HID, True)(
      gidx, dstp, xw1.reshape(_N * _R, _D_HID), zrow1, zdeg, ones)
  xw2, self2 = _tc_b(agg1[0], agg1[1], deg[0], deg[1], self1, wc2, W2_self)
  agg2 = _make_sc_agg(_D_OUT, False)(
      gidx, dstp, xw2.reshape(_N * _R, _D_OUT), zrow2)
  out = _tc_c(agg2[0], agg2[1], deg[0], deg[1], self2)
  return out


# speedup vs baseline: 2.4133x; 1.0939x over previous
import jax
import jax.numpy as jnp
from jax import lax
from jax.experimental import pallas as pl
from jax.experimental.pallas import tpu as pltpu
from jax.experimental.pallas import tpu_sc as plsc

_N = 10000
_E = 320000
_R = 8
_D_IN = 128
_D_HID = 64
_D_OUT = 32

_NC = 2
_NS = 16
_NW = _NC * _NS
_CHUNK = 128
_NCH = 80
_EPT = _NCH * _CHUNK
_E_PAD = _EPT * _NW
_N_ROWS = 10240
_ZR = _N_ROWS // _NS
_OR = _N_ROWS // _NS
_DEG_W = 8

_BN = 2000


def _make_sc_agg(d, with_deg):
  mesh = plsc.VectorSubcoreMesh(core_axis_name="c", subcore_axis_name="s")
  if with_deg:
    out_type = [jax.ShapeDtypeStruct((_NC, _N_ROWS, d), jnp.bfloat16),
                jax.ShapeDtypeStruct((_NC, _N_ROWS, _DEG_W), jnp.float32)]
  else:
    out_type = jax.ShapeDtypeStruct((_NC, _N_ROWS, d), jnp.bfloat16)
  scratch = [
      pltpu.VMEM((_NCH, _CHUNK), jnp.int32),
      pltpu.VMEM((_NCH, _CHUNK), jnp.int32),
      pltpu.VMEM((_CHUNK, d), jnp.bfloat16),
      pltpu.VMEM((_CHUNK, d), jnp.bfloat16),
      pltpu.VMEM_SHARED((_N_ROWS, d), jnp.bfloat16),
      pltpu.SemaphoreType.DMA,
      pltpu.SemaphoreType.DMA,
  ]
  if with_deg:
    scratch += [
        pltpu.VMEM((_CHUNK, _DEG_W), jnp.float32),
        pltpu.VMEM_SHARED((_N_ROWS, _DEG_W), jnp.float32),
    ]

  def body(*refs):
    if with_deg:
      (gidx_hbm, dst_hbm, table_hbm, zrow_hbm, zdeg_hbm, ones_hbm,
       out_hbm, deg_hbm,
       gidx_v, dst_v, buf0, buf1, agg_sh, sem0, sem1, ones_v, deg_sh) = refs
    else:
      (gidx_hbm, dst_hbm, table_hbm, zrow_hbm,
       out_hbm,
       gidx_v, dst_v, buf0, buf1, agg_sh, sem0, sem1) = refs
    c = lax.axis_index("c")
    s = lax.axis_index("s")
    wid = c * _NS + s

    pltpu.sync_copy(zrow_hbm, agg_sh.at[pl.ds(s * _ZR, _ZR)])
    if with_deg:
      pltpu.sync_copy(zdeg_hbm, deg_sh.at[pl.ds(s * _ZR, _ZR)])
      pltpu.sync_copy(ones_hbm, ones_v)
    pltpu.sync_copy(gidx_hbm.at[wid], gidx_v)
    pltpu.sync_copy(dst_hbm.at[wid], dst_v)
    plsc.subcore_barrier()

    def step(i, carry):
      d0 = pltpu.async_copy(table_hbm.at[gidx_v.at[2 * i]], buf0, sem0)
      d1 = pltpu.async_copy(table_hbm.at[gidx_v.at[2 * i + 1]], buf1, sem1)
      d0.wait()
      pltpu.sync_copy(buf0, agg_sh.at[dst_v.at[2 * i]], add=True)
      if with_deg:
        pltpu.sync_copy(ones_v, deg_sh.at[dst_v.at[2 * i]], add=True)
      d1.wait()
      pltpu.sync_copy(buf1, agg_sh.at[dst_v.at[2 * i + 1]], add=True)
      if with_deg:
        pltpu.sync_copy(ones_v, deg_sh.at[dst_v.at[2 * i + 1]], add=True)
      return carry
    lax.fori_loop(0, _NCH // 2, step, 0)

    plsc.subcore_barrier()
    pltpu.sync_copy(agg_sh.at[pl.ds(s * _OR, _OR)],
                    out_hbm.at[c].at[pl.ds(s * _OR, _OR)])
    if with_deg:
      pltpu.sync_copy(deg_sh.at[pl.ds(s * _OR, _OR)],
                      deg_hbm.at[c].at[pl.ds(s * _OR, _OR)])

  return pl.kernel(
      body, out_type=out_type, mesh=mesh, scratch_types=scratch,
      compiler_params=pltpu.CompilerParams(use_tc_tiling_on_sc=False))


def _tc_a_body(x_ref, wc_ref, ws_ref, xw_ref, sf_ref):
  xb = x_ref[...]
  xw = jnp.dot(xb, wc_ref[...], preferred_element_type=jnp.float32)
  xw_ref[...] = xw.astype(jnp.bfloat16)
  sf_ref[...] = jnp.dot(xb, ws_ref[...], preferred_element_type=jnp.float32)


_tc_a = pl.pallas_call(
    _tc_a_body,
    grid=(_N // _BN,),
    in_specs=[
        pl.BlockSpec((_BN, _D_IN), lambda i: (i, 0)),
        pl.BlockSpec((_D_IN, _R * _D_HID), lambda i: (0, 0)),
        pl.BlockSpec((_D_IN, _D_HID), lambda i: (0, 0)),
    ],
    out_specs=[
        pl.BlockSpec((_BN, _R * _D_HID), lambda i: (i, 0)),
        pl.BlockSpec((_BN, _D_HID), lambda i: (i, 0)),
    ],
    out_shape=[
        jax.ShapeDtypeStruct((_N, _R * _D_HID), jnp.bfloat16),
        jax.ShapeDtypeStruct((_N, _D_HID), jnp.float32),
    ],
)


def _tc_b_body(p0_ref, p1_ref, d0_ref, d1_ref, s1_ref, wc_ref, ws_ref,
               xw_ref, sf_ref):
  deg = jnp.maximum(d0_ref[:, 0:1] + d1_ref[:, 0:1], 1.0)
  p = p0_ref[...].astype(jnp.float32) + p1_ref[...].astype(jnp.float32)
  h = jnp.maximum(p / deg + s1_ref[...], 0.0)
  xw = jnp.dot(h, wc_ref[...], preferred_element_type=jnp.float32)
  xw_ref[...] = xw.astype(jnp.bfloat16)
  sf_ref[...] = jnp.dot(h, ws_ref[...], preferred_element_type=jnp.float32)


_tc_b = pl.pallas_call(
    _tc_b_body,
    grid=(_N // _BN,),
    in_specs=[
        pl.BlockSpec((_BN, _D_HID), lambda i: (i, 0)),
        pl.BlockSpec((_BN, _D_HID), lambda i: (i, 0)),
        pl.BlockSpec((_BN, _DEG_W), lambda i: (i, 0)),
        pl.BlockSpec((_BN, _DEG_W), lambda i: (i, 0)),
        pl.BlockSpec((_BN, _D_HID), lambda i: (i, 0)),
        pl.BlockSpec((_D_HID, _R * _D_OUT), lambda i: (0, 0)),
        pl.BlockSpec((_D_HID, _D_OUT), lambda i: (0, 0)),
    ],
    out_specs=[
        pl.BlockSpec((_BN, _R * _D_OUT), lambda i: (i, 0)),
        pl.BlockSpec((_BN, _D_OUT), lambda i: (i, 0)),
    ],
    out_shape=[
        jax.ShapeDtypeStruct((_N, _R * _D_OUT), jnp.bfloat16),
        jax.ShapeDtypeStruct((_N, _D_OUT), jnp.float32),
    ],
)


def _tc_c_body(q0_ref, q1_ref, d0_ref, d1_ref, s2_ref, out_ref):
  deg = jnp.maximum(d0_ref[:, 0:1] + d1_ref[:, 0:1], 1.0)
  q = q0_ref[...].astype(jnp.float32) + q1_ref[...].astype(jnp.float32)
  out_ref[...] = q / deg + s2_ref[...]


_tc_c = pl.pallas_call(
    _tc_c_body,
    grid=(_N // _BN,),
    in_specs=[
        pl.BlockSpec((_BN, _D_OUT), lambda i: (i, 0)),
        pl.BlockSpec((_BN, _D_OUT), lambda i: (i, 0)),
        pl.BlockSpec((_BN, _DEG_W), lambda i: (i, 0)),
        pl.BlockSpec((_BN, _DEG_W), lambda i: (i, 0)),
        pl.BlockSpec((_BN, _D_OUT), lambda i: (i, 0)),
    ],
    out_specs=pl.BlockSpec((_BN, _D_OUT), lambda i: (i, 0)),
    out_shape=jax.ShapeDtypeStruct((_N, _D_OUT), jnp.float32),
)


def kernel(x, edge_index, edge_type, W1_rel, W1_self, W2_rel, W2_self):
  src, dst = edge_index[0], edge_index[1]
  pad = _E_PAD - _E
  gidx = jnp.concatenate(
      [src * _R + edge_type, jnp.zeros((pad,), jnp.int32)]
  ).reshape(_NW, _NCH, _CHUNK)
  dstp = jnp.concatenate(
      [dst, jnp.full((pad,), _N, jnp.int32)]
  ).reshape(_NW, _NCH, _CHUNK)

  wc1 = W1_rel.transpose(1, 0, 2).reshape(_D_IN, _R * _D_HID)
  wc2 = W2_rel.transpose(1, 0, 2).reshape(_D_HID, _R * _D_OUT)

  zrow1 = jnp.zeros((_ZR, _D_HID), jnp.bfloat16)
  zrow2 = jnp.zeros((_ZR, _D_OUT), jnp.bfloat16)
  zdeg = jnp.zeros((_ZR, _DEG_W), jnp.float32)
  ones = jnp.ones((_CHUNK, _DEG_W), jnp.float32)

  xw1, self1 = _tc_a(x, wc1, W1_self)
  agg1, deg = _make_sc_agg(_D_HID, True)(
      gidx, dstp, xw1.reshape(_N * _R, _D_HID), zrow1, zdeg, ones)
  xw2, self2 = _tc_b(agg1[0], agg1[1], deg[0], deg[1], self1, wc2, W2_self)
  agg2 = _make_sc_agg(_D_OUT, False)(
      gidx, dstp, xw2.reshape(_N * _R, _D_OUT), zrow2)
  out = _tc_c(agg2[0], agg2[1], deg[0], deg[1], self2)
  return out


# speedup vs baseline: 2.4975x; 1.0349x over previous
import jax
import jax.numpy as jnp
from jax import lax
from jax.experimental import pallas as pl
from jax.experimental.pallas import tpu as pltpu
from jax.experimental.pallas import tpu_sc as plsc

_N = 10000
_E = 320000
_R = 8
_D_IN = 128
_D_HID = 64
_D_OUT = 32

_NC = 2
_NS = 16
_NW = _NC * _NS
_CHUNK = 128
_NCH = 80
_EPT = _NCH * _CHUNK
_E_PAD = _EPT * _NW
_N_ROWS = 10240
_ZR = _N_ROWS // _NS
_OR = _N_ROWS // _NS
_DEG_W = 8

_BN = 2000


def _make_sc_agg(d, with_deg):
  mesh = plsc.VectorSubcoreMesh(core_axis_name="c", subcore_axis_name="s")
  if with_deg:
    out_type = [jax.ShapeDtypeStruct((_NC, _N_ROWS, d), jnp.bfloat16),
                jax.ShapeDtypeStruct((_NC, _N_ROWS, _DEG_W), jnp.float32)]
  else:
    out_type = jax.ShapeDtypeStruct((_NC, _N_ROWS, d), jnp.bfloat16)
  scratch = [
      pltpu.VMEM((_NCH, _CHUNK), jnp.int32),
      pltpu.VMEM((_NCH, _CHUNK), jnp.int32),
      pltpu.VMEM((4, _CHUNK, d), jnp.bfloat16),
      pltpu.VMEM_SHARED((_N_ROWS, d), jnp.bfloat16),
      pltpu.SemaphoreType.DMA,
      pltpu.SemaphoreType.DMA,
      pltpu.SemaphoreType.DMA,
      pltpu.SemaphoreType.DMA,
  ]
  if with_deg:
    scratch += [
        pltpu.VMEM((_CHUNK, _DEG_W), jnp.float32),
        pltpu.VMEM_SHARED((_N_ROWS, _DEG_W), jnp.float32),
    ]

  def body(*refs):
    if with_deg:
      (gidx_hbm, dst_hbm, table_hbm, zrow_hbm, zdeg_hbm, ones_hbm,
       out_hbm, deg_hbm,
       gidx_v, dst_v, bufs, agg_sh, sem0, sem1, sem2, sem3,
       ones_v, deg_sh) = refs
    else:
      (gidx_hbm, dst_hbm, table_hbm, zrow_hbm,
       out_hbm,
       gidx_v, dst_v, bufs, agg_sh, sem0, sem1, sem2, sem3) = refs
    c = lax.axis_index("c")
    s = lax.axis_index("s")
    wid = c * _NS + s

    pltpu.sync_copy(zrow_hbm, agg_sh.at[pl.ds(s * _ZR, _ZR)])
    if with_deg:
      pltpu.sync_copy(zdeg_hbm, deg_sh.at[pl.ds(s * _ZR, _ZR)])
      pltpu.sync_copy(ones_hbm, ones_v)
    pltpu.sync_copy(gidx_hbm.at[wid], gidx_v)
    pltpu.sync_copy(dst_hbm.at[wid], dst_v)
    plsc.subcore_barrier()

    sems = (sem0, sem1, sem2, sem3)

    def step(i, carry):
      descs = [
          pltpu.async_copy(table_hbm.at[gidx_v.at[4 * i + j]],
                           bufs.at[j], sems[j])
          for j in range(4)
      ]
      for j in range(4):
        descs[j].wait()
        pltpu.sync_copy(bufs.at[j], agg_sh.at[dst_v.at[4 * i + j]], add=True)
        if with_deg:
          pltpu.sync_copy(ones_v, deg_sh.at[dst_v.at[4 * i + j]], add=True)
      return carry
    lax.fori_loop(0, _NCH // 4, step, 0)

    plsc.subcore_barrier()
    pltpu.sync_copy(agg_sh.at[pl.ds(s * _OR, _OR)],
                    out_hbm.at[c].at[pl.ds(s * _OR, _OR)])
    if with_deg:
      pltpu.sync_copy(deg_sh.at[pl.ds(s * _OR, _OR)],
                      deg_hbm.at[c].at[pl.ds(s * _OR, _OR)])

  return pl.kernel(
      body, out_type=out_type, mesh=mesh, scratch_types=scratch,
      compiler_params=pltpu.CompilerParams(use_tc_tiling_on_sc=False))


def _tc_a_body(x_ref, wc_ref, ws_ref, xw_ref, sf_ref):
  xb = x_ref[...]
  xw = jnp.dot(xb, wc_ref[...], preferred_element_type=jnp.float32)
  xw_ref[...] = xw.astype(jnp.bfloat16)
  sf_ref[...] = jnp.dot(xb, ws_ref[...], preferred_element_type=jnp.float32)


_tc_a = pl.pallas_call(
    _tc_a_body,
    grid=(_N // _BN,),
    in_specs=[
        pl.BlockSpec((_BN, _D_IN), lambda i: (i, 0)),
        pl.BlockSpec((_D_IN, _R * _D_HID), lambda i: (0, 0)),
        pl.BlockSpec((_D_IN, _D_HID), lambda i: (0, 0)),
    ],
    out_specs=[
        pl.BlockSpec((_BN, _R * _D_HID), lambda i: (i, 0)),
        pl.BlockSpec((_BN, _D_HID), lambda i: (i, 0)),
    ],
    out_shape=[
        jax.ShapeDtypeStruct((_N, _R * _D_HID), jnp.bfloat16),
        jax.ShapeDtypeStruct((_N, _D_HID), jnp.float32),
    ],
)


def _tc_b_body(p0_ref, p1_ref, d0_ref, d1_ref, s1_ref, wc_ref, ws_ref,
               xw_ref, sf_ref):
  deg = jnp.maximum(d0_ref[:, 0:1] + d1_ref[:, 0:1], 1.0)
  p = p0_ref[...].astype(jnp.float32) + p1_ref[...].astype(jnp.float32)
  h = jnp.maximum(p / deg + s1_ref[...], 0.0)
  xw = jnp.dot(h, wc_ref[...], preferred_element_type=jnp.float32)
  xw_ref[...] = xw.astype(jnp.bfloat16)
  sf_ref[...] = jnp.dot(h, ws_ref[...], preferred_element_type=jnp.float32)


_tc_b = pl.pallas_call(
    _tc_b_body,
    grid=(_N // _BN,),
    in_specs=[
        pl.BlockSpec((_BN, _D_HID), lambda i: (i, 0)),
        pl.BlockSpec((_BN, _D_HID), lambda i: (i, 0)),
        pl.BlockSpec((_BN, _DEG_W), lambda i: (i, 0)),
        pl.BlockSpec((_BN, _DEG_W), lambda i: (i, 0)),
        pl.BlockSpec((_BN, _D_HID), lambda i: (i, 0)),
        pl.BlockSpec((_D_HID, _R * _D_OUT), lambda i: (0, 0)),
        pl.BlockSpec((_D_HID, _D_OUT), lambda i: (0, 0)),
    ],
    out_specs=[
        pl.BlockSpec((_BN, _R * _D_OUT), lambda i: (i, 0)),
        pl.BlockSpec((_BN, _D_OUT), lambda i: (i, 0)),
    ],
    out_shape=[
        jax.ShapeDtypeStruct((_N, _R * _D_OUT), jnp.bfloat16),
        jax.ShapeDtypeStruct((_N, _D_OUT), jnp.float32),
    ],
)


def _tc_c_body(q0_ref, q1_ref, d0_ref, d1_ref, s2_ref, out_ref):
  deg = jnp.maximum(d0_ref[:, 0:1] + d1_ref[:, 0:1], 1.0)
  q = q0_ref[...].astype(jnp.float32) + q1_ref[...].astype(jnp.float32)
  out_ref[...] = q / deg + s2_ref[...]


_tc_c = pl.pallas_call(
    _tc_c_body,
    grid=(_N // _BN,),
    in_specs=[
        pl.BlockSpec((_BN, _D_OUT), lambda i: (i, 0)),
        pl.BlockSpec((_BN, _D_OUT), lambda i: (i, 0)),
        pl.BlockSpec((_BN, _DEG_W), lambda i: (i, 0)),
        pl.BlockSpec((_BN, _DEG_W), lambda i: (i, 0)),
        pl.BlockSpec((_BN, _D_OUT), lambda i: (i, 0)),
    ],
    out_specs=pl.BlockSpec((_BN, _D_OUT), lambda i: (i, 0)),
    out_shape=jax.ShapeDtypeStruct((_N, _D_OUT), jnp.float32),
)


def kernel(x, edge_index, edge_type, W1_rel, W1_self, W2_rel, W2_self):
  src, dst = edge_index[0], edge_index[1]
  pad = _E_PAD - _E
  gidx = jnp.concatenate(
      [src * _R + edge_type, jnp.zeros((pad,), jnp.int32)]
  ).reshape(_NW, _NCH, _CHUNK)
  dstp = jnp.concatenate(
      [dst, jnp.full((pad,), _N, jnp.int32)]
  ).reshape(_NW, _NCH, _CHUNK)

  wc1 = W1_rel.transpose(1, 0, 2).reshape(_D_IN, _R * _D_HID)
  wc2 = W2_rel.transpose(1, 0, 2).reshape(_D_HID, _R * _D_OUT)

  zrow1 = jnp.zeros((_ZR, _D_HID), jnp.bfloat16)
  zrow2 = jnp.zeros((_ZR, _D_OUT), jnp.bfloat16)
  zdeg = jnp.zeros((_ZR, _DEG_W), jnp.float32)
  ones = jnp.ones((_CHUNK, _DEG_W), jnp.float32)

  xw1, self1 = _tc_a(x, wc1, W1_self)
  agg1, deg = _make_sc_agg(_D_HID, True)(
      gidx, dstp, xw1.reshape(_N * _R, _D_HID), zrow1, zdeg, ones)
  xw2, self2 = _tc_b(agg1[0], agg1[1], deg[0], deg[1], self1, wc2, W2_self)
  agg2 = _make_sc_agg(_D_OUT, False)(
      gidx, dstp, xw2.reshape(_N * _R, _D_OUT), zrow2)
  out = _tc_c(agg2[0], agg2[1], deg[0], deg[1], self2)
  return out


# speedup vs baseline: 2.5418x; 1.0178x over previous
import jax
import jax.numpy as jnp
from jax import lax
from jax.experimental import pallas as pl
from jax.experimental.pallas import tpu as pltpu
from jax.experimental.pallas import tpu_sc as plsc

_N = 10000
_E = 320000
_R = 8
_D_IN = 128
_D_HID = 64
_D_OUT = 32

_NC = 2
_NS = 16
_NW = _NC * _NS
_CHUNK = 128
_NCH = 80
_EPT = _NCH * _CHUNK
_E_PAD = _EPT * _NW
_N_ROWS = 10240
_ZR = _N_ROWS // _NS
_OR = _N_ROWS // _NS
_DEG_W = 8

_BN = 2000


def _make_sc_agg(d, with_deg):
  mesh = plsc.VectorSubcoreMesh(core_axis_name="c", subcore_axis_name="s")
  if with_deg:
    out_type = [jax.ShapeDtypeStruct((_NC, _N_ROWS, d), jnp.bfloat16),
                jax.ShapeDtypeStruct((_NC, _N_ROWS, _DEG_W), jnp.float32)]
  else:
    out_type = jax.ShapeDtypeStruct((_NC, _N_ROWS, d), jnp.bfloat16)
  scratch = [
      pltpu.VMEM((_NCH, _CHUNK), jnp.int32),
      pltpu.VMEM((_NCH, _CHUNK), jnp.int32),
      pltpu.VMEM((4, _CHUNK, d), jnp.bfloat16),
      pltpu.VMEM_SHARED((_N_ROWS, d), jnp.bfloat16),
      pltpu.SemaphoreType.DMA,
      pltpu.SemaphoreType.DMA,
      pltpu.SemaphoreType.DMA,
      pltpu.SemaphoreType.DMA,
      pltpu.SemaphoreType.DMA,
      pltpu.SemaphoreType.DMA,
  ]
  if with_deg:
    scratch += [
        pltpu.VMEM((_CHUNK, _DEG_W), jnp.float32),
        pltpu.VMEM_SHARED((_N_ROWS, _DEG_W), jnp.float32),
    ]

  def body(*refs):
    if with_deg:
      (gidx_hbm, dst_hbm, table_hbm, zrow_hbm, zdeg_hbm, ones_hbm,
       out_hbm, deg_hbm,
       gidx_v, dst_v, bufs, agg_sh, sem0, sem1, sem2, sem3, sem_s, sem_d,
       ones_v, deg_sh) = refs
    else:
      (gidx_hbm, dst_hbm, table_hbm, zrow_hbm,
       out_hbm,
       gidx_v, dst_v, bufs, agg_sh, sem0, sem1, sem2, sem3,
       sem_s, sem_d) = refs
    c = lax.axis_index("c")
    s = lax.axis_index("s")
    wid = c * _NS + s

    pltpu.sync_copy(zrow_hbm, agg_sh.at[pl.ds(s * _ZR, _ZR)])
    if with_deg:
      pltpu.sync_copy(zdeg_hbm, deg_sh.at[pl.ds(s * _ZR, _ZR)])
      pltpu.sync_copy(ones_hbm, ones_v)
    pltpu.sync_copy(gidx_hbm.at[wid], gidx_v)
    pltpu.sync_copy(dst_hbm.at[wid], dst_v)
    plsc.subcore_barrier()

    sems = (sem0, sem1, sem2, sem3)

    def step(i, carry):
      gath = [
          pltpu.async_copy(table_hbm.at[gidx_v.at[4 * i + j]],
                           bufs.at[j], sems[j])
          for j in range(4)
      ]
      scat = []
      for j in range(4):
        gath[j].wait()
        scat.append(pltpu.async_copy(
            bufs.at[j], agg_sh.at[dst_v.at[4 * i + j]], sem_s, add=True))
        if with_deg:
          scat.append(pltpu.async_copy(
              ones_v, deg_sh.at[dst_v.at[4 * i + j]], sem_d, add=True))
      for dsc in scat:
        dsc.wait()
      return carry
    lax.fori_loop(0, _NCH // 4, step, 0)

    plsc.subcore_barrier()
    pltpu.sync_copy(agg_sh.at[pl.ds(s * _OR, _OR)],
                    out_hbm.at[c].at[pl.ds(s * _OR, _OR)])
    if with_deg:
      pltpu.sync_copy(deg_sh.at[pl.ds(s * _OR, _OR)],
                      deg_hbm.at[c].at[pl.ds(s * _OR, _OR)])

  return pl.kernel(
      body, out_type=out_type, mesh=mesh, scratch_types=scratch,
      compiler_params=pltpu.CompilerParams(use_tc_tiling_on_sc=False))


def _tc_a_body(x_ref, wc_ref, ws_ref, xw_ref, sf_ref):
  xb = x_ref[...]
  xw = jnp.dot(xb, wc_ref[...], preferred_element_type=jnp.float32)
  xw_ref[...] = xw.astype(jnp.bfloat16)
  sf_ref[...] = jnp.dot(xb, ws_ref[...], preferred_element_type=jnp.float32)


_tc_a = pl.pallas_call(
    _tc_a_body,
    grid=(_N // _BN,),
    in_specs=[
        pl.BlockSpec((_BN, _D_IN), lambda i: (i, 0)),
        pl.BlockSpec((_D_IN, _R * _D_HID), lambda i: (0, 0)),
        pl.BlockSpec((_D_IN, _D_HID), lambda i: (0, 0)),
    ],
    out_specs=[
        pl.BlockSpec((_BN, _R * _D_HID), lambda i: (i, 0)),
        pl.BlockSpec((_BN, _D_HID), lambda i: (i, 0)),
    ],
    out_shape=[
        jax.ShapeDtypeStruct((_N, _R * _D_HID), jnp.bfloat16),
        jax.ShapeDtypeStruct((_N, _D_HID), jnp.float32),
    ],
)


def _tc_b_body(p0_ref, p1_ref, d0_ref, d1_ref, s1_ref, wc_ref, ws_ref,
               xw_ref, sf_ref):
  deg = jnp.maximum(d0_ref[:, 0:1] + d1_ref[:, 0:1], 1.0)
  p = p0_ref[...].astype(jnp.float32) + p1_ref[...].astype(jnp.float32)
  h = jnp.maximum(p / deg + s1_ref[...], 0.0)
  xw = jnp.dot(h, wc_ref[...], preferred_element_type=jnp.float32)
  xw_ref[...] = xw.astype(jnp.bfloat16)
  sf_ref[...] = jnp.dot(h, ws_ref[...], preferred_element_type=jnp.float32)


_tc_b = pl.pallas_call(
    _tc_b_body,
    grid=(_N // _BN,),
    in_specs=[
        pl.BlockSpec((_BN, _D_HID), lambda i: (i, 0)),
        pl.BlockSpec((_BN, _D_HID), lambda i: (i, 0)),
        pl.BlockSpec((_BN, _DEG_W), lambda i: (i, 0)),
        pl.BlockSpec((_BN, _DEG_W), lambda i: (i, 0)),
        pl.BlockSpec((_BN, _D_HID), lambda i: (i, 0)),
        pl.BlockSpec((_D_HID, _R * _D_OUT), lambda i: (0, 0)),
        pl.BlockSpec((_D_HID, _D_OUT), lambda i: (0, 0)),
    ],
    out_specs=[
        pl.BlockSpec((_BN, _R * _D_OUT), lambda i: (i, 0)),
        pl.BlockSpec((_BN, _D_OUT), lambda i: (i, 0)),
    ],
    out_shape=[
        jax.ShapeDtypeStruct((_N, _R * _D_OUT), jnp.bfloat16),
        jax.ShapeDtypeStruct((_N, _D_OUT), jnp.float32),
    ],
)


def _tc_c_body(q0_ref, q1_ref, d0_ref, d1_ref, s2_ref, out_ref):
  deg = jnp.maximum(d0_ref[:, 0:1] + d1_ref[:, 0:1], 1.0)
  q = q0_ref[...].astype(jnp.float32) + q1_ref[...].astype(jnp.float32)
  out_ref[...] = q / deg + s2_ref[...]


_tc_c = pl.pallas_call(
    _tc_c_body,
    grid=(_N // _BN,),
    in_specs=[
        pl.BlockSpec((_BN, _D_OUT), lambda i: (i, 0)),
        pl.BlockSpec((_BN, _D_OUT), lambda i: (i, 0)),
        pl.BlockSpec((_BN, _DEG_W), lambda i: (i, 0)),
        pl.BlockSpec((_BN, _DEG_W), lambda i: (i, 0)),
        pl.BlockSpec((_BN, _D_OUT), lambda i: (i, 0)),
    ],
    out_specs=pl.BlockSpec((_BN, _D_OUT), lambda i: (i, 0)),
    out_shape=jax.ShapeDtypeStruct((_N, _D_OUT), jnp.float32),
)


def kernel(x, edge_index, edge_type, W1_rel, W1_self, W2_rel, W2_self):
  src, dst = edge_index[0], edge_index[1]
  pad = _E_PAD - _E
  gidx = jnp.concatenate(
      [src * _R + edge_type, jnp.zeros((pad,), jnp.int32)]
  ).reshape(_NW, _NCH, _CHUNK)
  dstp = jnp.concatenate(
      [dst, jnp.full((pad,), _N, jnp.int32)]
  ).reshape(_NW, _NCH, _CHUNK)

  wc1 = W1_rel.transpose(1, 0, 2).reshape(_D_IN, _R * _D_HID)
  wc2 = W2_rel.transpose(1, 0, 2).reshape(_D_HID, _R * _D_OUT)

  zrow1 = jnp.zeros((_ZR, _D_HID), jnp.bfloat16)
  zrow2 = jnp.zeros((_ZR, _D_OUT), jnp.bfloat16)
  zdeg = jnp.zeros((_ZR, _DEG_W), jnp.float32)
  ones = jnp.ones((_CHUNK, _DEG_W), jnp.float32)

  xw1, self1 = _tc_a(x, wc1, W1_self)
  agg1, deg = _make_sc_agg(_D_HID, True)(
      gidx, dstp, xw1.reshape(_N * _R, _D_HID), zrow1, zdeg, ones)
  xw2, self2 = _tc_b(agg1[0], agg1[1], deg[0], deg[1], self1, wc2, W2_self)
  agg2 = _make_sc_agg(_D_OUT, False)(
      gidx, dstp, xw2.reshape(_N * _R, _D_OUT), zrow2)
  out = _tc_c(agg2[0], agg2[1], deg[0], deg[1], self2)
  return out


# speedup vs baseline: 3.6899x; 1.4516x over previous
import jax
import jax.numpy as jnp
from jax import lax
from jax.experimental import pallas as pl
from jax.experimental.pallas import tpu as pltpu
from jax.experimental.pallas import tpu_sc as plsc

_N = 10000
_E = 320000
_R = 8
_D_IN = 128
_D_HID = 64
_D_OUT = 32

_NC = 2
_NS = 16
_NW = _NC * _NS
_CHUNK = 125
_NCH = 80
_EPT = _NCH * _CHUNK
_N_ROWS = 10240
_ZR = _N_ROWS // _NS
_OR = _N_ROWS // _NS
_DEG_W = 8

_BN = 2000


def _make_sc_agg(d, with_deg):
  mesh = plsc.VectorSubcoreMesh(core_axis_name="c", subcore_axis_name="s")
  if with_deg:
    out_type = [jax.ShapeDtypeStruct((_NC, _N_ROWS, d), jnp.bfloat16),
                jax.ShapeDtypeStruct((_NC, _N_ROWS, _DEG_W), jnp.float32)]
  else:
    out_type = jax.ShapeDtypeStruct((_NC, _N_ROWS, d), jnp.bfloat16)
  scratch = [
      pltpu.VMEM((_NCH, _CHUNK), jnp.int32),
      pltpu.VMEM((_NCH, _CHUNK), jnp.int32),
      pltpu.VMEM((4, _CHUNK, d), jnp.bfloat16),
      pltpu.VMEM_SHARED((_N_ROWS, d), jnp.bfloat16),
      pltpu.SemaphoreType.DMA,
      pltpu.SemaphoreType.DMA,
      pltpu.SemaphoreType.DMA,
      pltpu.SemaphoreType.DMA,
      pltpu.SemaphoreType.DMA,
      pltpu.SemaphoreType.DMA,
  ]
  if with_deg:
    scratch += [
        pltpu.VMEM((_CHUNK, _DEG_W), jnp.float32),
        pltpu.VMEM_SHARED((_N_ROWS, _DEG_W), jnp.float32),
    ]

  def body(*refs):
    if with_deg:
      (gidx_hbm, dst_hbm, table_hbm, zrow_hbm, zdeg_hbm, ones_hbm,
       out_hbm, deg_hbm,
       gidx_v, dst_v, bufs, agg_sh, sem0, sem1, sem2, sem3, sem_s, sem_d,
       ones_v, deg_sh) = refs
    else:
      (gidx_hbm, dst_hbm, table_hbm, zrow_hbm,
       out_hbm,
       gidx_v, dst_v, bufs, agg_sh, sem0, sem1, sem2, sem3,
       sem_s, sem_d) = refs
    c = lax.axis_index("c")
    s = lax.axis_index("s")
    wid = c * _NS + s

    pltpu.sync_copy(zrow_hbm, agg_sh.at[pl.ds(s * _ZR, _ZR)])
    if with_deg:
      pltpu.sync_copy(zdeg_hbm, deg_sh.at[pl.ds(s * _ZR, _ZR)])
      pltpu.sync_copy(ones_hbm, ones_v)
    pltpu.sync_copy(gidx_hbm.at[wid], gidx_v)
    pltpu.sync_copy(dst_hbm.at[wid], dst_v)
    plsc.subcore_barrier()

    sems = (sem0, sem1, sem2, sem3)

    def step(i, carry):
      gath = [
          pltpu.async_copy(table_hbm.at[gidx_v.at[4 * i + j]],
                           bufs.at[j], sems[j])
          for j in range(4)
      ]
      scat = []
      for j in range(4):
        gath[j].wait()
        scat.append(pltpu.async_copy(
            bufs.at[j], agg_sh.at[dst_v.at[4 * i + j]], sem_s, add=True))
        if with_deg:
          scat.append(pltpu.async_copy(
              ones_v, deg_sh.at[dst_v.at[4 * i + j]], sem_d, add=True))
      for dsc in scat:
        dsc.wait()
      return carry
    lax.fori_loop(0, _NCH // 4, step, 0)

    plsc.subcore_barrier()
    pltpu.sync_copy(agg_sh.at[pl.ds(s * _OR, _OR)],
                    out_hbm.at[c].at[pl.ds(s * _OR, _OR)])
    if with_deg:
      pltpu.sync_copy(deg_sh.at[pl.ds(s * _OR, _OR)],
                      deg_hbm.at[c].at[pl.ds(s * _OR, _OR)])

  return pl.kernel(
      body, out_type=out_type, mesh=mesh, scratch_types=scratch,
      compiler_params=pltpu.CompilerParams(use_tc_tiling_on_sc=False))


def _tc_a_body(x_ref, wc_ref, ws_ref, xw_ref, sf_ref):
  xb = x_ref[...]
  xw = jnp.dot(xb, wc_ref[...], preferred_element_type=jnp.float32)
  xw_ref[...] = xw.astype(jnp.bfloat16)
  sf_ref[...] = jnp.dot(xb, ws_ref[...], preferred_element_type=jnp.float32)


_tc_a = pl.pallas_call(
    _tc_a_body,
    grid=(_N // _BN,),
    in_specs=[
        pl.BlockSpec((_BN, _D_IN), lambda i: (i, 0)),
        pl.BlockSpec((_D_IN, _R * _D_HID), lambda i: (0, 0)),
        pl.BlockSpec((_D_IN, _D_HID), lambda i: (0, 0)),
    ],
    out_specs=[
        pl.BlockSpec((_BN, _R * _D_HID), lambda i: (i, 0)),
        pl.BlockSpec((_BN, _D_HID), lambda i: (i, 0)),
    ],
    out_shape=[
        jax.ShapeDtypeStruct((_N, _R * _D_HID), jnp.bfloat16),
        jax.ShapeDtypeStruct((_N, _D_HID), jnp.float32),
    ],
)


def _tc_b_body(p0_ref, p1_ref, d0_ref, d1_ref, s1_ref, wc_ref, ws_ref,
               xw_ref, sf_ref):
  deg = jnp.maximum(d0_ref[:, 0:1] + d1_ref[:, 0:1], 1.0)
  p = p0_ref[...].astype(jnp.float32) + p1_ref[...].astype(jnp.float32)
  h = jnp.maximum(p / deg + s1_ref[...], 0.0)
  xw = jnp.dot(h, wc_ref[...], preferred_element_type=jnp.float32)
  xw_ref[...] = xw.astype(jnp.bfloat16)
  sf_ref[...] = jnp.dot(h, ws_ref[...], preferred_element_type=jnp.float32)


_tc_b = pl.pallas_call(
    _tc_b_body,
    grid=(_N // _BN,),
    in_specs=[
        pl.BlockSpec((_BN, _D_HID), lambda i: (i, 0)),
        pl.BlockSpec((_BN, _D_HID), lambda i: (i, 0)),
        pl.BlockSpec((_BN, _DEG_W), lambda i: (i, 0)),
        pl.BlockSpec((_BN, _DEG_W), lambda i: (i, 0)),
        pl.BlockSpec((_BN, _D_HID), lambda i: (i, 0)),
        pl.BlockSpec((_D_HID, _R * _D_OUT), lambda i: (0, 0)),
        pl.BlockSpec((_D_HID, _D_OUT), lambda i: (0, 0)),
    ],
    out_specs=[
        pl.BlockSpec((_BN, _R * _D_OUT), lambda i: (i, 0)),
        pl.BlockSpec((_BN, _D_OUT), lambda i: (i, 0)),
    ],
    out_shape=[
        jax.ShapeDtypeStruct((_N, _R * _D_OUT), jnp.bfloat16),
        jax.ShapeDtypeStruct((_N, _D_OUT), jnp.float32),
    ],
)


def _tc_c_body(q0_ref, q1_ref, d0_ref, d1_ref, s2_ref, out_ref):
  deg = jnp.maximum(d0_ref[:, 0:1] + d1_ref[:, 0:1], 1.0)
  q = q0_ref[...].astype(jnp.float32) + q1_ref[...].astype(jnp.float32)
  out_ref[...] = q / deg + s2_ref[...]


_tc_c = pl.pallas_call(
    _tc_c_body,
    grid=(_N // _BN,),
    in_specs=[
        pl.BlockSpec((_BN, _D_OUT), lambda i: (i, 0)),
        pl.BlockSpec((_BN, _D_OUT), lambda i: (i, 0)),
        pl.BlockSpec((_BN, _DEG_W), lambda i: (i, 0)),
        pl.BlockSpec((_BN, _DEG_W), lambda i: (i, 0)),
        pl.BlockSpec((_BN, _D_OUT), lambda i: (i, 0)),
    ],
    out_specs=pl.BlockSpec((_BN, _D_OUT), lambda i: (i, 0)),
    out_shape=jax.ShapeDtypeStruct((_N, _D_OUT), jnp.float32),
)


def kernel(x, edge_index, edge_type, W1_rel, W1_self, W2_rel, W2_self):
  src, dst = edge_index[0], edge_index[1]
  gidx = (src * _R + edge_type).reshape(_NW, _NCH, _CHUNK)
  dstp = dst.reshape(_NW, _NCH, _CHUNK)

  wc1 = W1_rel.transpose(1, 0, 2).reshape(_D_IN, _R * _D_HID)
  wc2 = W2_rel.transpose(1, 0, 2).reshape(_D_HID, _R * _D_OUT)

  zrow1 = jnp.zeros((_ZR, _D_HID), jnp.bfloat16)
  zrow2 = jnp.zeros((_ZR, _D_OUT), jnp.bfloat16)
  zdeg = jnp.zeros((_ZR, _DEG_W), jnp.float32)
  ones = jnp.ones((_CHUNK, _DEG_W), jnp.float32)

  xw1, self1 = _tc_a(x, wc1, W1_self)
  agg1, deg = _make_sc_agg(_D_HID, True)(
      gidx, dstp, xw1.reshape(_N * _R, _D_HID), zrow1, zdeg, ones)
  xw2, self2 = _tc_b(agg1[0], agg1[1], deg[0], deg[1], self1, wc2, W2_self)
  agg2 = _make_sc_agg(_D_OUT, False)(
      gidx, dstp, xw2.reshape(_N * _R, _D_OUT), zrow2)
  out = _tc_c(agg2[0], agg2[1], deg[0], deg[1], self2)
  return out


# speedup vs baseline: 3.8954x; 1.0557x over previous
import jax
import jax.numpy as jnp
from jax import lax
from jax.experimental import pallas as pl
from jax.experimental.pallas import tpu as pltpu
from jax.experimental.pallas import tpu_sc as plsc

_N = 10000
_E = 320000
_R = 8
_D_IN = 128
_D_HID = 64
_D_OUT = 32

_NC = 2
_NS = 16
_NW = _NC * _NS
_CHUNK = 125
_NCH = 80
_EPT = _NCH * _CHUNK
_N_ROWS = 10240
_ZR = _N_ROWS // _NS
_OR = _N_ROWS // _NS
_DEG_W = 8

_BN = 2000


def _make_sc_agg(d, with_deg):
  mesh = plsc.VectorSubcoreMesh(core_axis_name="c", subcore_axis_name="s")
  if with_deg:
    out_type = [jax.ShapeDtypeStruct((_NC, _N_ROWS, d), jnp.bfloat16),
                jax.ShapeDtypeStruct((_NC, _N_ROWS, _DEG_W), jnp.float32)]
  else:
    out_type = jax.ShapeDtypeStruct((_NC, _N_ROWS, d), jnp.bfloat16)
  scratch = [
      pltpu.VMEM((_NCH, _CHUNK), jnp.int32),
      pltpu.VMEM((_NCH, _CHUNK), jnp.int32),
      pltpu.VMEM((8, _CHUNK, d), jnp.bfloat16),
      pltpu.VMEM_SHARED((_N_ROWS, d), jnp.bfloat16),
  ] + [pltpu.SemaphoreType.DMA] * 10
  if with_deg:
    scratch += [
        pltpu.VMEM((_CHUNK, _DEG_W), jnp.float32),
        pltpu.VMEM_SHARED((_N_ROWS, _DEG_W), jnp.float32),
    ]

  def body(*refs):
    if with_deg:
      (gidx_hbm, dst_hbm, table_hbm, zrow_hbm, zdeg_hbm, ones_hbm,
       out_hbm, deg_hbm,
       gidx_v, dst_v, bufs, agg_sh, *rest2) = refs
      sems, (sem_s, sem_d), (ones_v, deg_sh) = rest2[:8], rest2[8:10], rest2[10:]
    else:
      (gidx_hbm, dst_hbm, table_hbm, zrow_hbm,
       out_hbm,
       gidx_v, dst_v, bufs, agg_sh, *rest2) = refs
      sems, (sem_s, sem_d) = rest2[:8], rest2[8:10]
    c = lax.axis_index("c")
    s = lax.axis_index("s")
    wid = c * _NS + s

    pltpu.sync_copy(zrow_hbm, agg_sh.at[pl.ds(s * _ZR, _ZR)])
    if with_deg:
      pltpu.sync_copy(zdeg_hbm, deg_sh.at[pl.ds(s * _ZR, _ZR)])
      pltpu.sync_copy(ones_hbm, ones_v)
    pltpu.sync_copy(gidx_hbm.at[wid], gidx_v)
    pltpu.sync_copy(dst_hbm.at[wid], dst_v)
    plsc.subcore_barrier()

    def step(i, carry):
      gath = [
          pltpu.async_copy(table_hbm.at[gidx_v.at[8 * i + j]],
                           bufs.at[j], sems[j])
          for j in range(8)
      ]
      scat = []
      for j in range(8):
        gath[j].wait()
        scat.append(pltpu.async_copy(
            bufs.at[j], agg_sh.at[dst_v.at[8 * i + j]], sem_s, add=True))
        if with_deg:
          scat.append(pltpu.async_copy(
              ones_v, deg_sh.at[dst_v.at[8 * i + j]], sem_d, add=True))
      for dsc in scat:
        dsc.wait()
      return carry
    lax.fori_loop(0, _NCH // 8, step, 0)

    plsc.subcore_barrier()
    pltpu.sync_copy(agg_sh.at[pl.ds(s * _OR, _OR)],
                    out_hbm.at[c].at[pl.ds(s * _OR, _OR)])
    if with_deg:
      pltpu.sync_copy(deg_sh.at[pl.ds(s * _OR, _OR)],
                      deg_hbm.at[c].at[pl.ds(s * _OR, _OR)])

  return pl.kernel(
      body, out_type=out_type, mesh=mesh, scratch_types=scratch,
      compiler_params=pltpu.CompilerParams(use_tc_tiling_on_sc=False))


def _tc_a_body(x_ref, wc_ref, ws_ref, xw_ref, sf_ref):
  xb = x_ref[...]
  xw = jnp.dot(xb, wc_ref[...], preferred_element_type=jnp.float32)
  xw_ref[...] = xw.astype(jnp.bfloat16)
  sf_ref[...] = jnp.dot(xb, ws_ref[...], preferred_element_type=jnp.float32)


_tc_a = pl.pallas_call(
    _tc_a_body,
    grid=(_N // _BN,),
    in_specs=[
        pl.BlockSpec((_BN, _D_IN), lambda i: (i, 0)),
        pl.BlockSpec((_D_IN, _R * _D_HID), lambda i: (0, 0)),
        pl.BlockSpec((_D_IN, _D_HID), lambda i: (0, 0)),
    ],
    out_specs=[
        pl.BlockSpec((_BN, _R * _D_HID), lambda i: (i, 0)),
        pl.BlockSpec((_BN, _D_HID), lambda i: (i, 0)),
    ],
    out_shape=[
        jax.ShapeDtypeStruct((_N, _R * _D_HID), jnp.bfloat16),
        jax.ShapeDtypeStruct((_N, _D_HID), jnp.float32),
    ],
)


def _tc_b_body(p0_ref, p1_ref, d0_ref, d1_ref, s1_ref, wc_ref, ws_ref,
               xw_ref, sf_ref):
  deg = jnp.maximum(d0_ref[:, 0:1] + d1_ref[:, 0:1], 1.0)
  p = p0_ref[...].astype(jnp.float32) + p1_ref[...].astype(jnp.float32)
  h = jnp.maximum(p / deg + s1_ref[...], 0.0)
  xw = jnp.dot(h, wc_ref[...], preferred_element_type=jnp.float32)
  xw_ref[...] = xw.astype(jnp.bfloat16)
  sf_ref[...] = jnp.dot(h, ws_ref[...], preferred_element_type=jnp.float32)


_tc_b = pl.pallas_call(
    _tc_b_body,
    grid=(_N // _BN,),
    in_specs=[
        pl.BlockSpec((_BN, _D_HID), lambda i: (i, 0)),
        pl.BlockSpec((_BN, _D_HID), lambda i: (i, 0)),
        pl.BlockSpec((_BN, _DEG_W), lambda i: (i, 0)),
        pl.BlockSpec((_BN, _DEG_W), lambda i: (i, 0)),
        pl.BlockSpec((_BN, _D_HID), lambda i: (i, 0)),
        pl.BlockSpec((_D_HID, _R * _D_OUT), lambda i: (0, 0)),
        pl.BlockSpec((_D_HID, _D_OUT), lambda i: (0, 0)),
    ],
    out_specs=[
        pl.BlockSpec((_BN, _R * _D_OUT), lambda i: (i, 0)),
        pl.BlockSpec((_BN, _D_OUT), lambda i: (i, 0)),
    ],
    out_shape=[
        jax.ShapeDtypeStruct((_N, _R * _D_OUT), jnp.bfloat16),
        jax.ShapeDtypeStruct((_N, _D_OUT), jnp.float32),
    ],
)


def _tc_c_body(q0_ref, q1_ref, d0_ref, d1_ref, s2_ref, out_ref):
  deg = jnp.maximum(d0_ref[:, 0:1] + d1_ref[:, 0:1], 1.0)
  q = q0_ref[...].astype(jnp.float32) + q1_ref[...].astype(jnp.float32)
  out_ref[...] = q / deg + s2_ref[...]


_tc_c = pl.pallas_call(
    _tc_c_body,
    grid=(_N // _BN,),
    in_specs=[
        pl.BlockSpec((_BN, _D_OUT), lambda i: (i, 0)),
        pl.BlockSpec((_BN, _D_OUT), lambda i: (i, 0)),
        pl.BlockSpec((_BN, _DEG_W), lambda i: (i, 0)),
        pl.BlockSpec((_BN, _DEG_W), lambda i: (i, 0)),
        pl.BlockSpec((_BN, _D_OUT), lambda i: (i, 0)),
    ],
    out_specs=pl.BlockSpec((_BN, _D_OUT), lambda i: (i, 0)),
    out_shape=jax.ShapeDtypeStruct((_N, _D_OUT), jnp.float32),
)


def kernel(x, edge_index, edge_type, W1_rel, W1_self, W2_rel, W2_self):
  src, dst = edge_index[0], edge_index[1]
  gidx = (src * _R + edge_type).reshape(_NW, _NCH, _CHUNK)
  dstp = dst.reshape(_NW, _NCH, _CHUNK)

  wc1 = W1_rel.transpose(1, 0, 2).reshape(_D_IN, _R * _D_HID)
  wc2 = W2_rel.transpose(1, 0, 2).reshape(_D_HID, _R * _D_OUT)

  zrow1 = jnp.zeros((_ZR, _D_HID), jnp.bfloat16)
  zrow2 = jnp.zeros((_ZR, _D_OUT), jnp.bfloat16)
  zdeg = jnp.zeros((_ZR, _DEG_W), jnp.float32)
  ones = jnp.ones((_CHUNK, _DEG_W), jnp.float32)

  xw1, self1 = _tc_a(x, wc1, W1_self)
  agg1, deg = _make_sc_agg(_D_HID, True)(
      gidx, dstp, xw1.reshape(_N * _R, _D_HID), zrow1, zdeg, ones)
  xw2, self2 = _tc_b(agg1[0], agg1[1], deg[0], deg[1], self1, wc2, W2_self)
  agg2 = _make_sc_agg(_D_OUT, False)(
      gidx, dstp, xw2.reshape(_N * _R, _D_OUT), zrow2)
  out = _tc_c(agg2[0], agg2[1], deg[0], deg[1], self2)
  return out


# speedup vs baseline: 3.9527x; 1.0147x over previous
import jax
import jax.numpy as jnp
from jax import lax
from jax.experimental import pallas as pl
from jax.experimental.pallas import tpu as pltpu
from jax.experimental.pallas import tpu_sc as plsc

_N = 10000
_E = 320000
_R = 8
_D_IN = 128
_D_HID = 64
_D_OUT = 32

_NC = 2
_NS = 16
_NW = _NC * _NS
_CHUNK = 125
_NCH = 80
_EPT = _NCH * _CHUNK
_N_ROWS = 10240
_ZR = _N_ROWS // _NS
_OR = _N_ROWS // _NS
_DEG_W = 8

_BN = 2000


def _make_sc_agg(d, with_deg):
  mesh = plsc.VectorSubcoreMesh(core_axis_name="c", subcore_axis_name="s")
  if with_deg:
    out_type = [jax.ShapeDtypeStruct((_NC, _N_ROWS, d), jnp.bfloat16),
                jax.ShapeDtypeStruct((_NC, _N_ROWS, _DEG_W), jnp.float32)]
  else:
    out_type = jax.ShapeDtypeStruct((_NC, _N_ROWS, d), jnp.bfloat16)
  scratch = [
      pltpu.VMEM((_NCH, _CHUNK), jnp.int32),
      pltpu.VMEM((_NCH, _CHUNK), jnp.int32),
      pltpu.VMEM((10, _CHUNK, d), jnp.bfloat16),
      pltpu.VMEM_SHARED((_N_ROWS, d), jnp.bfloat16),
  ] + [pltpu.SemaphoreType.DMA] * 12
  if with_deg:
    scratch += [
        pltpu.VMEM((_CHUNK, _DEG_W), jnp.float32),
        pltpu.VMEM_SHARED((_N_ROWS, _DEG_W), jnp.float32),
    ]

  def body(*refs):
    if with_deg:
      (gidx_hbm, dst_hbm, table_hbm, zrow_hbm, zdeg_hbm, ones_hbm,
       out_hbm, deg_hbm,
       gidx_v, dst_v, bufs, agg_sh, *rest2) = refs
      sems, (sem_s, sem_d), (ones_v, deg_sh) = rest2[:10], rest2[10:12], rest2[12:]
    else:
      (gidx_hbm, dst_hbm, table_hbm, zrow_hbm,
       out_hbm,
       gidx_v, dst_v, bufs, agg_sh, *rest2) = refs
      sems, (sem_s, sem_d) = rest2[:10], rest2[10:12]
    c = lax.axis_index("c")
    s = lax.axis_index("s")
    wid = c * _NS + s

    pltpu.sync_copy(zrow_hbm, agg_sh.at[pl.ds(s * _ZR, _ZR)])
    if with_deg:
      pltpu.sync_copy(zdeg_hbm, deg_sh.at[pl.ds(s * _ZR, _ZR)])
      pltpu.sync_copy(ones_hbm, ones_v)
    pltpu.sync_copy(gidx_hbm.at[wid], gidx_v)
    pltpu.sync_copy(dst_hbm.at[wid], dst_v)
    plsc.subcore_barrier()

    def step(i, carry):
      gath = [
          pltpu.async_copy(table_hbm.at[gidx_v.at[10 * i + j]],
                           bufs.at[j], sems[j])
          for j in range(10)
      ]
      scat = []
      for j in range(10):
        gath[j].wait()
        scat.append(pltpu.async_copy(
            bufs.at[j], agg_sh.at[dst_v.at[10 * i + j]], sem_s, add=True))
        if with_deg:
          scat.append(pltpu.async_copy(
              ones_v, deg_sh.at[dst_v.at[10 * i + j]], sem_d, add=True))
      for dsc in scat:
        dsc.wait()
      return carry
    lax.fori_loop(0, _NCH // 10, step, 0)

    plsc.subcore_barrier()
    pltpu.sync_copy(agg_sh.at[pl.ds(s * _OR, _OR)],
                    out_hbm.at[c].at[pl.ds(s * _OR, _OR)])
    if with_deg:
      pltpu.sync_copy(deg_sh.at[pl.ds(s * _OR, _OR)],
                      deg_hbm.at[c].at[pl.ds(s * _OR, _OR)])

  return pl.kernel(
      body, out_type=out_type, mesh=mesh, scratch_types=scratch,
      compiler_params=pltpu.CompilerParams(use_tc_tiling_on_sc=False))


def _tc_a_body(x_ref, wc_ref, ws_ref, xw_ref, sf_ref):
  xb = x_ref[...]
  xw = jnp.dot(xb, wc_ref[...], preferred_element_type=jnp.float32)
  xw_ref[...] = xw.astype(jnp.bfloat16)
  sf_ref[...] = jnp.dot(xb, ws_ref[...], preferred_element_type=jnp.float32)


_tc_a = pl.pallas_call(
    _tc_a_body,
    grid=(_N // _BN,),
    in_specs=[
        pl.BlockSpec((_BN, _D_IN), lambda i: (i, 0)),
        pl.BlockSpec((_D_IN, _R * _D_HID), lambda i: (0, 0)),
        pl.BlockSpec((_D_IN, _D_HID), lambda i: (0, 0)),
    ],
    out_specs=[
        pl.BlockSpec((_BN, _R * _D_HID), lambda i: (i, 0)),
        pl.BlockSpec((_BN, _D_HID), lambda i: (i, 0)),
    ],
    out_shape=[
        jax.ShapeDtypeStruct((_N, _R * _D_HID), jnp.bfloat16),
        jax.ShapeDtypeStruct((_N, _D_HID), jnp.float32),
    ],
)


def _tc_b_body(p0_ref, p1_ref, d0_ref, d1_ref, s1_ref, wc_ref, ws_ref,
               xw_ref, sf_ref):
  deg = jnp.maximum(d0_ref[:, 0:1] + d1_ref[:, 0:1], 1.0)
  p = p0_ref[...].astype(jnp.float32) + p1_ref[...].astype(jnp.float32)
  h = jnp.maximum(p / deg + s1_ref[...], 0.0)
  xw = jnp.dot(h, wc_ref[...], preferred_element_type=jnp.float32)
  xw_ref[...] = xw.astype(jnp.bfloat16)
  sf_ref[...] = jnp.dot(h, ws_ref[...], preferred_element_type=jnp.float32)


_tc_b = pl.pallas_call(
    _tc_b_body,
    grid=(_N // _BN,),
    in_specs=[
        pl.BlockSpec((_BN, _D_HID), lambda i: (i, 0)),
        pl.BlockSpec((_BN, _D_HID), lambda i: (i, 0)),
        pl.BlockSpec((_BN, _DEG_W), lambda i: (i, 0)),
        pl.BlockSpec((_BN, _DEG_W), lambda i: (i, 0)),
        pl.BlockSpec((_BN, _D_HID), lambda i: (i, 0)),
        pl.BlockSpec((_D_HID, _R * _D_OUT), lambda i: (0, 0)),
        pl.BlockSpec((_D_HID, _D_OUT), lambda i: (0, 0)),
    ],
    out_specs=[
        pl.BlockSpec((_BN, _R * _D_OUT), lambda i: (i, 0)),
        pl.BlockSpec((_BN, _D_OUT), lambda i: (i, 0)),
    ],
    out_shape=[
        jax.ShapeDtypeStruct((_N, _R * _D_OUT), jnp.bfloat16),
        jax.ShapeDtypeStruct((_N, _D_OUT), jnp.float32),
    ],
)


def _tc_c_body(q0_ref, q1_ref, d0_ref, d1_ref, s2_ref, out_ref):
  deg = jnp.maximum(d0_ref[:, 0:1] + d1_ref[:, 0:1], 1.0)
  q = q0_ref[...].astype(jnp.float32) + q1_ref[...].astype(jnp.float32)
  out_ref[...] = q / deg + s2_ref[...]


_tc_c = pl.pallas_call(
    _tc_c_body,
    grid=(_N // _BN,),
    in_specs=[
        pl.BlockSpec((_BN, _D_OUT), lambda i: (i, 0)),
        pl.BlockSpec((_BN, _D_OUT), lambda i: (i, 0)),
        pl.BlockSpec((_BN, _DEG_W), lambda i: (i, 0)),
        pl.BlockSpec((_BN, _DEG_W), lambda i: (i, 0)),
        pl.BlockSpec((_BN, _D_OUT), lambda i: (i, 0)),
    ],
    out_specs=pl.BlockSpec((_BN, _D_OUT), lambda i: (i, 0)),
    out_shape=jax.ShapeDtypeStruct((_N, _D_OUT), jnp.float32),
)


def kernel(x, edge_index, edge_type, W1_rel, W1_self, W2_rel, W2_self):
  src, dst = edge_index[0], edge_index[1]
  gidx = (src * _R + edge_type).reshape(_NW, _NCH, _CHUNK)
  dstp = dst.reshape(_NW, _NCH, _CHUNK)

  wc1 = W1_rel.transpose(1, 0, 2).reshape(_D_IN, _R * _D_HID)
  wc2 = W2_rel.transpose(1, 0, 2).reshape(_D_HID, _R * _D_OUT)

  zrow1 = jnp.zeros((_ZR, _D_HID), jnp.bfloat16)
  zrow2 = jnp.zeros((_ZR, _D_OUT), jnp.bfloat16)
  zdeg = jnp.zeros((_ZR, _DEG_W), jnp.float32)
  ones = jnp.ones((_CHUNK, _DEG_W), jnp.float32)

  xw1, self1 = _tc_a(x, wc1, W1_self)
  agg1, deg = _make_sc_agg(_D_HID, True)(
      gidx, dstp, xw1.reshape(_N * _R, _D_HID), zrow1, zdeg, ones)
  xw2, self2 = _tc_b(agg1[0], agg1[1], deg[0], deg[1], self1, wc2, W2_self)
  agg2 = _make_sc_agg(_D_OUT, False)(
      gidx, dstp, xw2.reshape(_N * _R, _D_OUT), zrow2)
  out = _tc_c(agg2[0], agg2[1], deg[0], deg[1], self2)
  return out
